# baseline (speedup 1.0000x reference)
import functools

import jax
import jax.numpy as jnp
from jax import lax
from jax.experimental import pallas as pl
from jax.experimental.pallas import tpu as pltpu
from jax.experimental.pallas import tpu_sc as plsc

EMB = 64
BATCH = 16384

_info = plsc.get_sparse_core_info()
NC, NS, L = _info.num_cores, _info.num_subcores, _info.num_lanes
NW = NC * NS
BPW = BATCH // NW

_mesh = plsc.VectorSubcoreMesh(core_axis_name="c", subcore_axis_name="s")


@functools.partial(
    pl.kernel,
    mesh=_mesh,
    out_type=jax.ShapeDtypeStruct((BATCH, EMB), jnp.float32),
    scratch_types=[
        pltpu.VMEM((BPW,), jnp.int32),
        pltpu.VMEM((BPW,), jnp.int32),
        pltpu.VMEM((BPW, EMB), jnp.float32),
        pltpu.VMEM((BPW, EMB), jnp.float32),
        pltpu.SemaphoreType.DMA,
        pltpu.SemaphoreType.DMA,
    ],
    compiler_params=pltpu.CompilerParams(use_tc_tiling_on_sc=False),
)
def _path_encoder(idx_cur_hbm, idx_last_hbm, table_hbm, out_hbm,
                  idx_c_v, idx_l_v, rows_c, rows_l, sem_c, sem_l):
    wid = lax.axis_index("s") * NC + lax.axis_index("c")
    base = wid * BPW
    pltpu.sync_copy(idx_cur_hbm.at[pl.ds(base, BPW)], idx_c_v)
    pltpu.sync_copy(idx_last_hbm.at[pl.ds(base, BPW)], idx_l_v)
    cp_c = pltpu.async_copy(table_hbm.at[idx_c_v], rows_c, sem_c)
    cp_l = pltpu.async_copy(table_hbm.at[idx_l_v], rows_l, sem_l)
    cp_c.wait()
    cp_l.wait()

    def body(i, carry):
        for c in range(EMB // L):
            a = rows_c[i, pl.ds(c * L, L)]
            b = rows_l[i, pl.ds(c * L, L)]
            rows_c[i, pl.ds(c * L, L)] = a * b
        return carry

    lax.fori_loop(0, BPW, body, 0)
    pltpu.sync_copy(rows_c, out_hbm.at[pl.ds(base, BPW)])


def kernel(actionList, table):
    idx = actionList.astype(jnp.int32)
    return _path_encoder(idx[:, 1], idx[:, 0], table)


# speedup vs baseline: 1.7016x; 1.7016x over previous
import functools

import jax
import jax.numpy as jnp
from jax import lax
from jax.experimental import pallas as pl
from jax.experimental.pallas import tpu as pltpu
from jax.experimental.pallas import tpu_sc as plsc

EMB = 64
BATCH = 16384

_info = plsc.get_sparse_core_info()
NC, NS, L = _info.num_cores, _info.num_subcores, _info.num_lanes
NW = NC * NS
BPW = BATCH // NW
CHUNK = 256
NCH = BPW // CHUNK

_mesh = plsc.VectorSubcoreMesh(core_axis_name="c", subcore_axis_name="s")


@functools.partial(
    pl.kernel,
    mesh=_mesh,
    out_type=jax.ShapeDtypeStruct((BATCH, EMB), jnp.float32),
    scratch_types=[
        pltpu.VMEM((BPW,), jnp.int32),
        pltpu.VMEM((BPW,), jnp.int32),
        pltpu.VMEM((CHUNK, EMB), jnp.float32),
        pltpu.VMEM((CHUNK, EMB), jnp.float32),
        pltpu.SemaphoreType.DMA,
    ],
)
def _path_encoder(idx_cur_hbm, idx_last_hbm, table_hbm, out_hbm,
                  idx_c_v, idx_l_v, rows_c, rows_l, sem):
    wid = lax.axis_index("s") * NC + lax.axis_index("c")
    base = wid * BPW
    pltpu.sync_copy(idx_cur_hbm.at[pl.ds(base, BPW)], idx_c_v)
    pltpu.sync_copy(idx_last_hbm.at[pl.ds(base, BPW)], idx_l_v)

    def chunk_body(ch, carry):
        off = ch * CHUNK

        def fire(g, carry):
            start = pl.multiple_of(off + g * L, L)
            vals_c = idx_c_v[pl.ds(start, L)]
            vals_l = idx_l_v[pl.ds(start, L)]
            for j in range(L):
                pltpu.async_copy(
                    table_hbm.at[pl.ds(vals_c[j], 1)],
                    rows_c.at[pl.ds(g * L + j, 1)], sem)
                pltpu.async_copy(
                    table_hbm.at[pl.ds(vals_l[j], 1)],
                    rows_l.at[pl.ds(g * L + j, 1)], sem)
            return carry

        lax.fori_loop(0, CHUNK // L, fire, 0)
        pltpu.make_async_copy(table_hbm.at[pl.ds(0, CHUNK)], rows_c, sem).wait()
        pltpu.make_async_copy(table_hbm.at[pl.ds(0, CHUNK)], rows_l, sem).wait()

        def mul(i, carry):
            for c in range(EMB // L):
                a = rows_c[i, pl.ds(c * L, L)]
                b = rows_l[i, pl.ds(c * L, L)]
                rows_c[i, pl.ds(c * L, L)] = a * b
            return carry

        lax.fori_loop(0, CHUNK, mul, 0)
        pltpu.sync_copy(rows_c, out_hbm.at[pl.ds(base + off, CHUNK)])
        return carry

    lax.fori_loop(0, NCH, chunk_body, 0)


def kernel(actionList, table):
    idx = actionList.astype(jnp.int32)
    return _path_encoder(idx[:, 1], idx[:, 0], table)
